# Initial kernel scaffold; baseline (speedup 1.0000x reference)
import jax
import jax.numpy as jnp
from jax import lax
from jax.experimental import pallas as pl
from jax.experimental.pallas import tpu as pltpu
from jax.experimental.pallas import tpu_sc as plsc

N = 10000
E = 320000
D = 128
R = 3

NC = 2
NS = 16
NW = NC * NS
EPW = E // NW
K = 128
NFULL = EPW // K
TAIL = EPW - NFULL * K
CNT = R * N
CNT_PAD = 30720
CPT = CNT_PAD // NS
RPT = N // NS

_mesh = plsc.VectorSubcoreMesh(core_axis_name="c", subcore_axis_name="s")


def _worker():
    cid = lax.axis_index("c")
    sid = lax.axis_index("s")
    return cid, sid, sid * NC + cid


def _pass_a_body(src_hbm, dst_hbm, typ_hbm, cnt2_hbm, g_hbm, c_hbm,
                 src_v, dst_v, typ_v, g_v, c_v, ones_v, zb_v,
                 s16, d16, t16, g16, c16, o16, cnt_sh):
    cid, sid, wid = _worker()
    base = wid * EPW

    @pl.loop(0, CPT // 16)
    def _zero_zb(i):
        zb_v[pl.ds(i * 16, 16)] = jnp.zeros((16,), jnp.float32)

    @pl.loop(0, K // 16)
    def _ones(i):
        ones_v[pl.ds(i * 16, 16)] = jnp.full((16,), 1.0, jnp.float32)

    @pl.loop(0, 1)
    def _ones16(i):
        o16[pl.ds(0, 16)] = jnp.full((16,), 1.0, jnp.float32)

    pltpu.sync_copy(zb_v, cnt_sh.at[pl.ds(sid * CPT, CPT)])
    plsc.subcore_barrier()

    def do_chunk(off, k, sv, dv, tv, gv, cv, ov):
        pltpu.sync_copy(src_hbm.at[pl.ds(off, k)], sv)
        pltpu.sync_copy(dst_hbm.at[pl.ds(off, k)], dv)
        pltpu.sync_copy(typ_hbm.at[pl.ds(off, k)], tv)

        @pl.loop(0, k // 16)
        def _compute(j):
            sl = pl.ds(j * 16, 16)
            t = tv[sl] * N
            gv[sl] = t + sv[sl]
            cv[sl] = t + dv[sl]

        pltpu.sync_copy(gv, g_hbm.at[pl.ds(off, k)])
        pltpu.sync_copy(cv, c_hbm.at[pl.ds(off, k)])
        pltpu.sync_copy(ov, cnt_sh.at[cv], add=True)

    @pl.loop(0, NFULL)
    def _chunks(i):
        do_chunk(base + i * K, K, src_v, dst_v, typ_v, g_v, c_v, ones_v)

    do_chunk(base + NFULL * K, TAIL, s16, d16, t16, g16, c16, o16)

    plsc.subcore_barrier()
    pltpu.sync_copy(cnt_sh.at[pl.ds(sid * CPT, CPT)],
                    cnt2_hbm.at[cid, pl.ds(sid * CPT, CPT)])


_pass_a = pl.kernel(
    _pass_a_body,
    out_type=(
        jax.ShapeDtypeStruct((NC, CNT_PAD), jnp.float32),
        jax.ShapeDtypeStruct((E,), jnp.int32),
        jax.ShapeDtypeStruct((E,), jnp.int32),
    ),
    mesh=_mesh,
    scratch_types=[
        pltpu.VMEM((K,), jnp.int32),
        pltpu.VMEM((K,), jnp.int32),
        pltpu.VMEM((K,), jnp.int32),
        pltpu.VMEM((K,), jnp.int32),
        pltpu.VMEM((K,), jnp.int32),
        pltpu.VMEM((K,), jnp.float32),
        pltpu.VMEM((CPT,), jnp.float32),
        pltpu.VMEM((16,), jnp.int32),
        pltpu.VMEM((16,), jnp.int32),
        pltpu.VMEM((16,), jnp.int32),
        pltpu.VMEM((16,), jnp.int32),
        pltpu.VMEM((16,), jnp.int32),
        pltpu.VMEM((16,), jnp.float32),
        pltpu.VMEM_SHARED((CNT_PAD,), jnp.float32),
    ],
)


def _pass_b_body(cnt2_hbm, c_hbm, w_hbm,
                 a_v, b_v, inv_v, c_v, w_v, c16, w16):
    cid, sid, wid = _worker()
    base = wid * EPW

    pltpu.sync_copy(cnt2_hbm.at[0], a_v)
    pltpu.sync_copy(cnt2_hbm.at[1], b_v)

    @pl.loop(0, CNT_PAD // 16)
    def _inv(i):
        sl = pl.ds(i * 16, 16)
        tot = a_v[sl] + b_v[sl]
        inv_v[sl] = 1.0 / jnp.maximum(tot, 1.0)

    def do_chunk(off, k, cv, wv):
        pltpu.sync_copy(c_hbm.at[pl.ds(off, k)], cv)

        @pl.loop(0, k // 16)
        def _gather(j):
            sl = pl.ds(j * 16, 16)
            wv[sl] = plsc.load_gather(inv_v, [cv[sl]])

        pltpu.sync_copy(wv, w_hbm.at[pl.ds(off, k)])

    @pl.loop(0, NFULL)
    def _chunks(i):
        do_chunk(base + i * K, K, c_v, w_v)

    do_chunk(base + NFULL * K, TAIL, c16, w16)


_pass_b = pl.kernel(
    _pass_b_body,
    out_type=jax.ShapeDtypeStruct((E,), jnp.float32),
    mesh=_mesh,
    scratch_types=[
        pltpu.VMEM((CNT_PAD,), jnp.float32),
        pltpu.VMEM((CNT_PAD,), jnp.float32),
        pltpu.VMEM((CNT_PAD,), jnp.float32),
        pltpu.VMEM((K,), jnp.int32),
        pltpu.VMEM((K,), jnp.float32),
        pltpu.VMEM((16,), jnp.int32),
        pltpu.VMEM((16,), jnp.float32),
    ],
)


def _pass_m_body(y_hbm, g_hbm, d_hbm, w_hbm, acc2_hbm,
                 g_v, d_v, w_v, rows_v, g16, d16, w16, rows16, sem, acc_sh):
    cid, sid, wid = _worker()
    base = wid * EPW

    @pl.loop(0, K)
    def _zero_rows(i):
        for j in range(D // 16):
            rows_v[i, pl.ds(j * 16, 16)] = jnp.zeros((16,), jnp.float32)

    for k in range(5):
        pltpu.sync_copy(rows_v.at[pl.ds(0, 125)],
                        acc_sh.at[pl.ds(sid * RPT + k * 125, 125)])
    plsc.subcore_barrier()

    def do_chunk(off, k, gv, dv, wv, rv):
        pltpu.sync_copy(g_hbm.at[pl.ds(off, k)], gv)
        pltpu.sync_copy(d_hbm.at[pl.ds(off, k)], dv)
        pltpu.sync_copy(w_hbm.at[pl.ds(off, k)], wv)
        pltpu.async_copy(y_hbm.at[gv], rv, sem).wait()

        @pl.loop(0, k)
        def _scale(e):
            we = wv[e]
            for j in range(D // 16):
                sl = pl.ds(j * 16, 16)
                rv[e, sl] = rv[e, sl] * we

        pltpu.sync_copy(rv, acc_sh.at[dv], add=True)

    @pl.loop(0, NFULL)
    def _chunks(i):
        do_chunk(base + i * K, K, g_v, d_v, w_v, rows_v)

    do_chunk(base + NFULL * K, TAIL, g16, d16, w16, rows16)

    plsc.subcore_barrier()
    for k in range(5):
        sl = pl.ds(sid * RPT + k * 125, 125)
        pltpu.sync_copy(acc_sh.at[sl], acc2_hbm.at[cid, sl])


_pass_m = pl.kernel(
    _pass_m_body,
    out_type=jax.ShapeDtypeStruct((NC, N, D), jnp.float32),
    mesh=_mesh,
    scratch_types=[
        pltpu.VMEM((K,), jnp.int32),
        pltpu.VMEM((K,), jnp.int32),
        pltpu.VMEM((K,), jnp.float32),
        pltpu.VMEM((K, D), jnp.float32),
        pltpu.VMEM((16,), jnp.int32),
        pltpu.VMEM((16,), jnp.int32),
        pltpu.VMEM((16,), jnp.float32),
        pltpu.VMEM((16, D), jnp.float32),
        pltpu.SemaphoreType.DMA,
        pltpu.VMEM_SHARED((N, D), jnp.float32),
    ],
)


BLK = 1000
NB = N // BLK


def _mm(a, b):
    return jnp.dot(a, b, preferred_element_type=jnp.float32)


def _k_in_body(x_ref, win_ref, bin_ref, wcat_ref, out_ref):
    h = jnp.maximum(_mm(x_ref[...], win_ref[...]) + bin_ref[0][None, :], 0.0)
    for r in range(R + 1):
        out_ref[r] = _mm(h, wcat_ref[r])


_k_in = pl.pallas_call(
    _k_in_body,
    grid=(NB,),
    in_specs=[
        pl.BlockSpec((BLK, D), lambda i: (i, 0)),
        pl.BlockSpec((D, D), lambda i: (0, 0)),
        pl.BlockSpec((1, D), lambda i: (0, 0)),
        pl.BlockSpec((R + 1, D, D), lambda i: (0, 0, 0)),
    ],
    out_specs=pl.BlockSpec((R + 1, BLK, D), lambda i: (0, i, 0)),
    out_shape=jax.ShapeDtypeStruct((R + 1, N, D), jnp.float32),
)


def _k_comb_body(root_ref, a0_ref, a1_ref, b_ref, wcat_ref, out_ref):
    h = jnp.maximum(
        root_ref[...] + a0_ref[...] + a1_ref[...] + b_ref[0][None, :], 0.0)
    for r in range(R + 1):
        out_ref[r] = _mm(h, wcat_ref[r])


_k_comb = pl.pallas_call(
    _k_comb_body,
    grid=(NB,),
    in_specs=[
        pl.BlockSpec((BLK, D), lambda i: (i, 0)),
        pl.BlockSpec((BLK, D), lambda i: (i, 0)),
        pl.BlockSpec((BLK, D), lambda i: (i, 0)),
        pl.BlockSpec((1, D), lambda i: (0, 0)),
        pl.BlockSpec((R + 1, D, D), lambda i: (0, 0, 0)),
    ],
    out_specs=pl.BlockSpec((R + 1, BLK, D), lambda i: (0, i, 0)),
    out_shape=jax.ShapeDtypeStruct((R + 1, N, D), jnp.float32),
)


def _k_mlp_body(root_ref, a0_ref, a1_ref, b_ref,
                wo1_ref, bo1_ref, wo2_ref, bo2_ref, wo3_ref, bo3_ref,
                out_ref):
    h = jnp.maximum(
        root_ref[...] + a0_ref[...] + a1_ref[...] + b_ref[0][None, :], 0.0)
    o = jnp.maximum(_mm(h, wo1_ref[...]) + bo1_ref[0][None, :], 0.0)
    o = jnp.maximum(_mm(o, wo2_ref[...]) + bo2_ref[0][None, :], 0.0)
    out_ref[...] = _mm(o, wo3_ref[...]) + bo3_ref[0][None, :]


_k_mlp = pl.pallas_call(
    _k_mlp_body,
    grid=(NB,),
    in_specs=[
        pl.BlockSpec((BLK, D), lambda i: (i, 0)),
        pl.BlockSpec((BLK, D), lambda i: (i, 0)),
        pl.BlockSpec((BLK, D), lambda i: (i, 0)),
        pl.BlockSpec((1, D), lambda i: (0, 0)),
        pl.BlockSpec((D, 512), lambda i: (0, 0)),
        pl.BlockSpec((1, 512), lambda i: (0, 0)),
        pl.BlockSpec((512, 256), lambda i: (0, 0)),
        pl.BlockSpec((1, 256), lambda i: (0, 0)),
        pl.BlockSpec((256, 128), lambda i: (0, 0)),
        pl.BlockSpec((1, 128), lambda i: (0, 0)),
    ],
    out_specs=pl.BlockSpec((BLK, 128), lambda i: (i, 0)),
    out_shape=jax.ShapeDtypeStruct((N, 128), jnp.float32),
)


def kernel(x, edge_index, edge_type, W_in, b_in, w1_rel, w1_root, b1,
           w2_rel, w2_root, b2, w3_rel, w3_root, b3,
           Wo1, bo1, Wo2, bo2, Wo3, bo3):
    src = edge_index[0]
    dst = edge_index[1]

    cnt2, g, c = _pass_a(src, dst, edge_type)
    w = _pass_b(cnt2, c)

    wcat1 = jnp.concatenate([w1_rel, w1_root[None]], axis=0)
    wcat2 = jnp.concatenate([w2_rel, w2_root[None]], axis=0)
    wcat3 = jnp.concatenate([w3_rel, w3_root[None]], axis=0)

    y = _k_in(x, W_in, b_in.reshape(1, D), wcat1)
    acc = _pass_m(y.reshape((R + 1) * N, D), g, dst, w)
    y = _k_comb(y[R], acc[0], acc[1], b1.reshape(1, D), wcat2)
    acc = _pass_m(y.reshape((R + 1) * N, D), g, dst, w)
    y = _k_comb(y[R], acc[0], acc[1], b2.reshape(1, D), wcat3)
    acc = _pass_m(y.reshape((R + 1) * N, D), g, dst, w)

    wo3p = jnp.pad(Wo3, ((0, 0), (0, 128 - Wo3.shape[1])))
    bo3p = jnp.pad(bo3, (0, 128 - bo3.shape[0]))
    out = _k_mlp(y[R], acc[0], acc[1], b3.reshape(1, D),
                 Wo1, bo1.reshape(1, 512), Wo2, bo2.reshape(1, 256),
                 wo3p, bo3p.reshape(1, 128))
    return out[:, :Wo3.shape[1]]


# speedup vs baseline: 6.9300x; 6.9300x over previous
import jax
import jax.numpy as jnp
from jax import lax
from jax.experimental import pallas as pl
from jax.experimental.pallas import tpu as pltpu
from jax.experimental.pallas import tpu_sc as plsc

N = 10000
E = 320000
D = 128
R = 3

NC = 2
NS = 16
NW = NC * NS
EPW = E // NW
K = 128
NFULL = EPW // K
TAIL = EPW - NFULL * K
CNT = R * N
CNT_PAD = 30720
CPT = CNT_PAD // NS
RPT = N // NS

_mesh = plsc.VectorSubcoreMesh(core_axis_name="c", subcore_axis_name="s")


def _worker():
    cid = lax.axis_index("c")
    sid = lax.axis_index("s")
    return cid, sid, sid * NC + cid


def _pass_a_body(src_hbm, dst_hbm, typ_hbm, cnt2_hbm, g_hbm, c_hbm,
                 src_v, dst_v, typ_v, g_v, c_v, ones_v, zb_v,
                 s16, d16, t16, g16, c16, o16, cnt_sh):
    cid, sid, wid = _worker()
    base = wid * EPW

    @pl.loop(0, CPT // 16)
    def _zero_zb(i):
        zb_v[pl.ds(i * 16, 16)] = jnp.zeros((16,), jnp.float32)

    @pl.loop(0, K // 16)
    def _ones(i):
        ones_v[pl.ds(i * 16, 16)] = jnp.full((16,), 1.0, jnp.float32)

    @pl.loop(0, 1)
    def _ones16(i):
        o16[pl.ds(0, 16)] = jnp.full((16,), 1.0, jnp.float32)

    pltpu.sync_copy(zb_v, cnt_sh.at[pl.ds(sid * CPT, CPT)])
    plsc.subcore_barrier()

    def do_chunk(off, k, sv, dv, tv, gv, cv, ov):
        pltpu.sync_copy(src_hbm.at[pl.ds(off, k)], sv)
        pltpu.sync_copy(dst_hbm.at[pl.ds(off, k)], dv)
        pltpu.sync_copy(typ_hbm.at[pl.ds(off, k)], tv)

        @pl.loop(0, k // 16)
        def _compute(j):
            sl = pl.ds(j * 16, 16)
            t = tv[sl] * N
            gv[sl] = t + sv[sl]
            cv[sl] = t + dv[sl]

        pltpu.sync_copy(gv, g_hbm.at[pl.ds(off, k)])
        pltpu.sync_copy(cv, c_hbm.at[pl.ds(off, k)])
        pltpu.sync_copy(ov, cnt_sh.at[cv], add=True)

    @pl.loop(0, NFULL)
    def _chunks(i):
        do_chunk(base + i * K, K, src_v, dst_v, typ_v, g_v, c_v, ones_v)

    do_chunk(base + NFULL * K, TAIL, s16, d16, t16, g16, c16, o16)

    plsc.subcore_barrier()
    pltpu.sync_copy(cnt_sh.at[pl.ds(sid * CPT, CPT)],
                    cnt2_hbm.at[cid, 0, pl.ds(sid * CPT, CPT)])


_pass_a = pl.kernel(
    _pass_a_body,
    out_type=(
        jax.ShapeDtypeStruct((NC, 1, CNT_PAD), jnp.float32),
        jax.ShapeDtypeStruct((E,), jnp.int32),
        jax.ShapeDtypeStruct((E,), jnp.int32),
    ),
    mesh=_mesh,
    compiler_params=pltpu.CompilerParams(needs_layout_passes=False),
    scratch_types=[
        pltpu.VMEM((K,), jnp.int32),
        pltpu.VMEM((K,), jnp.int32),
        pltpu.VMEM((K,), jnp.int32),
        pltpu.VMEM((K,), jnp.int32),
        pltpu.VMEM((K,), jnp.int32),
        pltpu.VMEM((K,), jnp.float32),
        pltpu.VMEM((CPT,), jnp.float32),
        pltpu.VMEM((16,), jnp.int32),
        pltpu.VMEM((16,), jnp.int32),
        pltpu.VMEM((16,), jnp.int32),
        pltpu.VMEM((16,), jnp.int32),
        pltpu.VMEM((16,), jnp.int32),
        pltpu.VMEM((16,), jnp.float32),
        pltpu.VMEM_SHARED((CNT_PAD,), jnp.float32),
    ],
)


def _pass_b_body(cnt2_hbm, c_hbm, w_hbm,
                 a_v, b_v, inv_v, c_v, w_v, c16, w16):
    cid, sid, wid = _worker()
    base = wid * EPW

    pltpu.sync_copy(cnt2_hbm.at[0, 0], a_v)
    pltpu.sync_copy(cnt2_hbm.at[1, 0], b_v)

    @pl.loop(0, CNT_PAD // 16)
    def _inv(i):
        sl = pl.ds(i * 16, 16)
        tot = a_v[sl] + b_v[sl]
        inv_v[sl] = 1.0 / jnp.maximum(tot, 1.0)

    def do_chunk(off, k, cv, wv):
        pltpu.sync_copy(c_hbm.at[pl.ds(off, k)], cv)

        @pl.loop(0, k // 16)
        def _gather(j):
            sl = pl.ds(j * 16, 16)
            wv[sl] = plsc.load_gather(inv_v, [cv[sl]])

        pltpu.sync_copy(wv, w_hbm.at[pl.ds(off, k)])

    @pl.loop(0, NFULL)
    def _chunks(i):
        do_chunk(base + i * K, K, c_v, w_v)

    do_chunk(base + NFULL * K, TAIL, c16, w16)


_pass_b = pl.kernel(
    _pass_b_body,
    out_type=jax.ShapeDtypeStruct((E,), jnp.float32),
    mesh=_mesh,
    compiler_params=pltpu.CompilerParams(needs_layout_passes=False),
    scratch_types=[
        pltpu.VMEM((CNT_PAD,), jnp.float32),
        pltpu.VMEM((CNT_PAD,), jnp.float32),
        pltpu.VMEM((CNT_PAD,), jnp.float32),
        pltpu.VMEM((K,), jnp.int32),
        pltpu.VMEM((K,), jnp.float32),
        pltpu.VMEM((16,), jnp.int32),
        pltpu.VMEM((16,), jnp.float32),
    ],
)


def _pass_m_body(y_hbm, g_hbm, d_hbm, w_hbm, acc2_hbm,
                 g_v, d_v, w_v, rows_v, g16, d16, w16, rows16, sem, acc_sh):
    cid, sid, wid = _worker()
    base = wid * EPW

    @pl.loop(0, K)
    def _zero_rows(i):
        for j in range(D // 16):
            rows_v[i, pl.ds(j * 16, 16)] = jnp.zeros((16,), jnp.float32)

    for k in range(5):
        idx = sid * 5 + k

        @pl.when(idx < (N // K))
        def _zero_chunk():
            off = pl.multiple_of(idx * K, K)
            pltpu.sync_copy(rows_v, acc_sh.at[pl.ds(off, K)])

    @pl.when(sid == NS - 1)
    def _zero_tail():
        pltpu.sync_copy(rows_v.at[pl.ds(0, 16)],
                        acc_sh.at[pl.ds((N // K) * K, 16)])

    plsc.subcore_barrier()

    def do_chunk(off, k, gv, dv, wv, rv):
        pltpu.sync_copy(g_hbm.at[pl.ds(off, k)], gv)
        pltpu.sync_copy(d_hbm.at[pl.ds(off, k)], dv)
        pltpu.sync_copy(w_hbm.at[pl.ds(off, k)], wv)
        pltpu.async_copy(y_hbm.at[gv], rv, sem).wait()

        @pl.loop(0, k)
        def _scale(e):
            we = plsc.load_gather(wv, [jnp.zeros((16,), jnp.int32) + e])
            for j in range(D // 16):
                sl = pl.ds(j * 16, 16)
                rv[e, sl] = rv[e, sl] * we

        pltpu.sync_copy(rv, acc_sh.at[dv], add=True)

    @pl.loop(0, NFULL)
    def _chunks(i):
        do_chunk(base + i * K, K, g_v, d_v, w_v, rows_v)

    do_chunk(base + NFULL * K, TAIL, g16, d16, w16, rows16)

    plsc.subcore_barrier()
    for k in range(5):
        idx = sid * 5 + k

        @pl.when(idx < (N // K))
        def _dump_chunk():
            off = pl.multiple_of(idx * K, K)
            pltpu.sync_copy(acc_sh.at[pl.ds(off, K)],
                            acc2_hbm.at[cid, pl.ds(off, K)])

    @pl.when(sid == NS - 1)
    def _dump_tail():
        sl = pl.ds((N // K) * K, 16)
        pltpu.sync_copy(acc_sh.at[sl], acc2_hbm.at[cid, sl])


_pass_m = pl.kernel(
    _pass_m_body,
    out_type=jax.ShapeDtypeStruct((NC, N, D), jnp.float32),
    mesh=_mesh,
    compiler_params=pltpu.CompilerParams(needs_layout_passes=False),
    scratch_types=[
        pltpu.VMEM((K,), jnp.int32),
        pltpu.VMEM((K,), jnp.int32),
        pltpu.VMEM((K,), jnp.float32),
        pltpu.VMEM((K, D), jnp.float32),
        pltpu.VMEM((16,), jnp.int32),
        pltpu.VMEM((16,), jnp.int32),
        pltpu.VMEM((16,), jnp.float32),
        pltpu.VMEM((16, D), jnp.float32),
        pltpu.SemaphoreType.DMA,
        pltpu.VMEM_SHARED((N, D), jnp.float32),
    ],
)


BLK = 1000
NB = N // BLK


def _mm(a, b):
    return jnp.dot(a, b, preferred_element_type=jnp.float32,
                   precision=jax.lax.Precision.HIGHEST)


def _k_in_body(x_ref, win_ref, bin_ref, wcat_ref, out_ref):
    h = jnp.maximum(_mm(x_ref[...], win_ref[...]) + bin_ref[0][None, :], 0.0)
    for r in range(R + 1):
        out_ref[r] = _mm(h, wcat_ref[r])


_k_in = pl.pallas_call(
    _k_in_body,
    grid=(NB,),
    in_specs=[
        pl.BlockSpec((BLK, D), lambda i: (i, 0)),
        pl.BlockSpec((D, D), lambda i: (0, 0)),
        pl.BlockSpec((1, D), lambda i: (0, 0)),
        pl.BlockSpec((R + 1, D, D), lambda i: (0, 0, 0)),
    ],
    out_specs=pl.BlockSpec((R + 1, BLK, D), lambda i: (0, i, 0)),
    out_shape=jax.ShapeDtypeStruct((R + 1, N, D), jnp.float32),
)


def _k_comb_body(root_ref, a0_ref, a1_ref, b_ref, wcat_ref, out_ref):
    h = jnp.maximum(
        root_ref[...] + a0_ref[...] + a1_ref[...] + b_ref[0][None, :], 0.0)
    for r in range(R + 1):
        out_ref[r] = _mm(h, wcat_ref[r])


_k_comb = pl.pallas_call(
    _k_comb_body,
    grid=(NB,),
    in_specs=[
        pl.BlockSpec((BLK, D), lambda i: (i, 0)),
        pl.BlockSpec((BLK, D), lambda i: (i, 0)),
        pl.BlockSpec((BLK, D), lambda i: (i, 0)),
        pl.BlockSpec((1, D), lambda i: (0, 0)),
        pl.BlockSpec((R + 1, D, D), lambda i: (0, 0, 0)),
    ],
    out_specs=pl.BlockSpec((R + 1, BLK, D), lambda i: (0, i, 0)),
    out_shape=jax.ShapeDtypeStruct((R + 1, N, D), jnp.float32),
)


def _k_mlp_body(root_ref, a0_ref, a1_ref, b_ref,
                wo1_ref, bo1_ref, wo2_ref, bo2_ref, wo3_ref, bo3_ref,
                out_ref):
    h = jnp.maximum(
        root_ref[...] + a0_ref[...] + a1_ref[...] + b_ref[0][None, :], 0.0)
    o = jnp.maximum(_mm(h, wo1_ref[...]) + bo1_ref[0][None, :], 0.0)
    o = jnp.maximum(_mm(o, wo2_ref[...]) + bo2_ref[0][None, :], 0.0)
    out_ref[...] = _mm(o, wo3_ref[...]) + bo3_ref[0][None, :]


_k_mlp = pl.pallas_call(
    _k_mlp_body,
    grid=(NB,),
    in_specs=[
        pl.BlockSpec((BLK, D), lambda i: (i, 0)),
        pl.BlockSpec((BLK, D), lambda i: (i, 0)),
        pl.BlockSpec((BLK, D), lambda i: (i, 0)),
        pl.BlockSpec((1, D), lambda i: (0, 0)),
        pl.BlockSpec((D, 512), lambda i: (0, 0)),
        pl.BlockSpec((1, 512), lambda i: (0, 0)),
        pl.BlockSpec((512, 256), lambda i: (0, 0)),
        pl.BlockSpec((1, 256), lambda i: (0, 0)),
        pl.BlockSpec((256, 128), lambda i: (0, 0)),
        pl.BlockSpec((1, 128), lambda i: (0, 0)),
    ],
    out_specs=pl.BlockSpec((BLK, 128), lambda i: (i, 0)),
    out_shape=jax.ShapeDtypeStruct((N, 128), jnp.float32),
)


def kernel(x, edge_index, edge_type, W_in, b_in, w1_rel, w1_root, b1,
           w2_rel, w2_root, b2, w3_rel, w3_root, b3,
           Wo1, bo1, Wo2, bo2, Wo3, bo3):
    src = edge_index[0]
    dst = edge_index[1]

    cnt2, g, c = _pass_a(src, dst, edge_type)
    w = _pass_b(cnt2, c)

    wcat1 = jnp.concatenate([w1_rel, w1_root[None]], axis=0)
    wcat2 = jnp.concatenate([w2_rel, w2_root[None]], axis=0)
    wcat3 = jnp.concatenate([w3_rel, w3_root[None]], axis=0)

    y = _k_in(x, W_in, b_in.reshape(1, D), wcat1)
    acc = _pass_m(y.reshape((R + 1) * N, D), g, dst, w)
    y = _k_comb(y[R], acc[0], acc[1], b1.reshape(1, D), wcat2)
    acc = _pass_m(y.reshape((R + 1) * N, D), g, dst, w)
    y = _k_comb(y[R], acc[0], acc[1], b2.reshape(1, D), wcat3)
    acc = _pass_m(y.reshape((R + 1) * N, D), g, dst, w)

    wo3p = jnp.pad(Wo3, ((0, 0), (0, 128 - Wo3.shape[1])))
    bo3p = jnp.pad(bo3, (0, 128 - bo3.shape[0]))
    out = _k_mlp(y[R], acc[0], acc[1], b3.reshape(1, D),
                 Wo1, bo1.reshape(1, 512), Wo2, bo2.reshape(1, 256),
                 wo3p, bo3p.reshape(1, 128))
    return out[:, :Wo3.shape[1]]


# speedup vs baseline: 14.5186x; 2.0950x over previous
import jax
import jax.numpy as jnp
from jax import lax
from jax.experimental import pallas as pl
from jax.experimental.pallas import tpu as pltpu
from jax.experimental.pallas import tpu_sc as plsc

N = 10000
E = 320000
D = 128
R = 3

NC = 2
NS = 16
NW = NC * NS
EPW = E // NW
K = 128
NFULL = EPW // K
TAIL = EPW - NFULL * K
CNT = R * N
CNT_PAD = 30720
CPT = CNT_PAD // NS
RPT = N // NS

_mesh = plsc.VectorSubcoreMesh(core_axis_name="c", subcore_axis_name="s")


def _worker():
    cid = lax.axis_index("c")
    sid = lax.axis_index("s")
    return cid, sid, sid * NC + cid


def _pass_a_body(src_hbm, dst_hbm, typ_hbm, cnt2_hbm, g_hbm, c_hbm,
                 src_v, dst_v, typ_v, g_v, c_v, ones_v, zb_v,
                 s16, d16, t16, g16, c16, o16, cnt_sh):
    cid, sid, wid = _worker()
    base = wid * EPW

    @pl.loop(0, CPT // 16)
    def _zero_zb(i):
        zb_v[pl.ds(i * 16, 16)] = jnp.zeros((16,), jnp.float32)

    @pl.loop(0, K // 16)
    def _ones(i):
        ones_v[pl.ds(i * 16, 16)] = jnp.full((16,), 1.0, jnp.float32)

    @pl.loop(0, 1)
    def _ones16(i):
        o16[pl.ds(0, 16)] = jnp.full((16,), 1.0, jnp.float32)

    pltpu.sync_copy(zb_v, cnt_sh.at[pl.ds(sid * CPT, CPT)])
    plsc.subcore_barrier()

    def do_chunk(off, k, sv, dv, tv, gv, cv, ov):
        pltpu.sync_copy(src_hbm.at[pl.ds(off, k)], sv)
        pltpu.sync_copy(dst_hbm.at[pl.ds(off, k)], dv)
        pltpu.sync_copy(typ_hbm.at[pl.ds(off, k)], tv)

        @pl.loop(0, k // 16)
        def _compute(j):
            sl = pl.ds(j * 16, 16)
            t = tv[sl] * N
            gv[sl] = t + sv[sl]
            cv[sl] = t + dv[sl]

        pltpu.sync_copy(gv, g_hbm.at[pl.ds(off, k)])
        pltpu.sync_copy(cv, c_hbm.at[pl.ds(off, k)])
        pltpu.sync_copy(ov, cnt_sh.at[cv], add=True)

    @pl.loop(0, NFULL)
    def _chunks(i):
        do_chunk(base + i * K, K, src_v, dst_v, typ_v, g_v, c_v, ones_v)

    do_chunk(base + NFULL * K, TAIL, s16, d16, t16, g16, c16, o16)

    plsc.subcore_barrier()
    pltpu.sync_copy(cnt_sh.at[pl.ds(sid * CPT, CPT)],
                    cnt2_hbm.at[cid, 0, pl.ds(sid * CPT, CPT)])


_pass_a = pl.kernel(
    _pass_a_body,
    out_type=(
        jax.ShapeDtypeStruct((NC, 1, CNT_PAD), jnp.float32),
        jax.ShapeDtypeStruct((E,), jnp.int32),
        jax.ShapeDtypeStruct((E,), jnp.int32),
    ),
    mesh=_mesh,
    compiler_params=pltpu.CompilerParams(needs_layout_passes=False),
    scratch_types=[
        pltpu.VMEM((K,), jnp.int32),
        pltpu.VMEM((K,), jnp.int32),
        pltpu.VMEM((K,), jnp.int32),
        pltpu.VMEM((K,), jnp.int32),
        pltpu.VMEM((K,), jnp.int32),
        pltpu.VMEM((K,), jnp.float32),
        pltpu.VMEM((CPT,), jnp.float32),
        pltpu.VMEM((16,), jnp.int32),
        pltpu.VMEM((16,), jnp.int32),
        pltpu.VMEM((16,), jnp.int32),
        pltpu.VMEM((16,), jnp.int32),
        pltpu.VMEM((16,), jnp.int32),
        pltpu.VMEM((16,), jnp.float32),
        pltpu.VMEM_SHARED((CNT_PAD,), jnp.float32),
    ],
)


def _pass_b_body(cnt2_hbm, c_hbm, w_hbm,
                 a_v, b_v, inv_v, c_v, w_v, c16, w16):
    cid, sid, wid = _worker()
    base = wid * EPW

    pltpu.sync_copy(cnt2_hbm.at[0, 0], a_v)
    pltpu.sync_copy(cnt2_hbm.at[1, 0], b_v)

    @pl.loop(0, CNT_PAD // 16)
    def _inv(i):
        sl = pl.ds(i * 16, 16)
        tot = a_v[sl] + b_v[sl]
        inv_v[sl] = 1.0 / jnp.maximum(tot, 1.0)

    def do_chunk(off, k, cv, wv):
        pltpu.sync_copy(c_hbm.at[pl.ds(off, k)], cv)

        @pl.loop(0, k // 16)
        def _gather(j):
            sl = pl.ds(j * 16, 16)
            wv[sl] = plsc.load_gather(inv_v, [cv[sl]])

        pltpu.sync_copy(wv, w_hbm.at[pl.ds(off, k)])

    @pl.loop(0, NFULL)
    def _chunks(i):
        do_chunk(base + i * K, K, c_v, w_v)

    do_chunk(base + NFULL * K, TAIL, c16, w16)


_pass_b = pl.kernel(
    _pass_b_body,
    out_type=jax.ShapeDtypeStruct((E,), jnp.float32),
    mesh=_mesh,
    compiler_params=pltpu.CompilerParams(needs_layout_passes=False),
    scratch_types=[
        pltpu.VMEM((CNT_PAD,), jnp.float32),
        pltpu.VMEM((CNT_PAD,), jnp.float32),
        pltpu.VMEM((CNT_PAD,), jnp.float32),
        pltpu.VMEM((K,), jnp.int32),
        pltpu.VMEM((K,), jnp.float32),
        pltpu.VMEM((16,), jnp.int32),
        pltpu.VMEM((16,), jnp.float32),
    ],
)


CH = 40
NCHW = EPW // CH
NSLOT = 5
NZC = N // CH


def _pass_m_body(y_hbm, g_hbm, d_hbm, w_hbm, acc2_hbm,
                 gbufs, dbufs, wbufs, rows, gsems, isems, ssems, acc_sh):
    cid, sid, wid = _worker()
    base = wid * EPW

    def idx_start(c, s):
        off = pl.ds(base + c * CH, CH)
        pltpu.async_copy(g_hbm.at[off], gbufs[s], isems[s])
        pltpu.async_copy(d_hbm.at[off], dbufs[s], isems[s])
        pltpu.async_copy(w_hbm.at[off], wbufs[s], isems[s])

    def idx_wait(c, s):
        off = pl.ds(base + c * CH, CH)
        pltpu.make_async_copy(g_hbm.at[off], gbufs[s], isems[s]).wait()
        pltpu.make_async_copy(d_hbm.at[off], dbufs[s], isems[s]).wait()
        pltpu.make_async_copy(w_hbm.at[off], wbufs[s], isems[s]).wait()

    def scat_wait(s):
        pltpu.make_async_copy(rows[s], acc_sh.at[dbufs[s]], ssems[s]).wait()

    @pl.loop(0, CH)
    def _zero_rows(i):
        for j in range(D // 16):
            rows[0][i, pl.ds(j * 16, 16)] = jnp.zeros((16,), jnp.float32)

    for k in range(16):
        zc = sid + NS * k

        @pl.when(zc < NZC)
        def _zero_chunk():
            off = pl.multiple_of(zc * CH, CH)
            pltpu.sync_copy(rows[0], acc_sh.at[pl.ds(off, CH)])

    plsc.subcore_barrier()

    for c in range(3):
        idx_start(c, c)
    for c in range(2):
        idx_wait(c, c)
        pltpu.async_copy(y_hbm.at[gbufs[c]], rows[c], gsems[c])

    @pl.loop(0, NCHW // NSLOT)
    def _visits(i):
        for b in range(NSLOT):
            c = i * NSLOT + b

            s3 = (b + 3) % NSLOT
            c3 = c + 3

            @pl.when(c3 < NCHW)
            def _issue_idx():
                @pl.when(c3 >= NSLOT)
                def _drain_scat():
                    scat_wait(s3)

                idx_start(c3, s3)

            s2 = (b + 2) % NSLOT
            c2 = c + 2

            @pl.when(c2 < NCHW)
            def _issue_gather():
                idx_wait(c2, s2)
                pltpu.async_copy(y_hbm.at[gbufs[s2]], rows[s2], gsems[s2])

            pltpu.make_async_copy(y_hbm.at[gbufs[b]], rows[b],
                                  gsems[b]).wait()

            @pl.loop(0, CH)
            def _scale(e):
                we = plsc.load_gather(
                    wbufs[b], [jnp.zeros((16,), jnp.int32) + e])
                for j in range(D // 16):
                    sl = pl.ds(j * 16, 16)
                    rows[b][e, sl] = rows[b][e, sl] * we

            pltpu.async_copy(rows[b], acc_sh.at[dbufs[b]], ssems[b],
                             add=True)

    for s in range(NSLOT):
        scat_wait(s)

    plsc.subcore_barrier()
    for k in range(16):
        zc = sid + NS * k

        @pl.when(zc < NZC)
        def _dump_chunk():
            off = pl.multiple_of(zc * CH, CH)
            pltpu.sync_copy(acc_sh.at[pl.ds(off, CH)],
                            acc2_hbm.at[cid, pl.ds(off, CH)])


_pass_m = pl.kernel(
    _pass_m_body,
    out_type=jax.ShapeDtypeStruct((NC, N, D), jnp.float32),
    mesh=_mesh,
    compiler_params=pltpu.CompilerParams(needs_layout_passes=False),
    scratch_types=[
        [pltpu.VMEM((CH,), jnp.int32) for _ in range(NSLOT)],
        [pltpu.VMEM((CH,), jnp.int32) for _ in range(NSLOT)],
        [pltpu.VMEM((CH,), jnp.float32) for _ in range(NSLOT)],
        [pltpu.VMEM((CH, D), jnp.float32) for _ in range(NSLOT)],
        [pltpu.SemaphoreType.DMA for _ in range(NSLOT)],
        [pltpu.SemaphoreType.DMA for _ in range(NSLOT)],
        [pltpu.SemaphoreType.DMA for _ in range(NSLOT)],
        pltpu.VMEM_SHARED((N, D), jnp.float32),
    ],
)


BLK = 1000
NB = N // BLK


def _mm(a, b):
    return jnp.dot(a, b, preferred_element_type=jnp.float32)


def _k_in_body(x_ref, win_ref, bin_ref, wcat_ref, out_ref):
    h = jnp.maximum(_mm(x_ref[...], win_ref[...]) + bin_ref[0][None, :], 0.0)
    for r in range(R + 1):
        out_ref[r] = _mm(h, wcat_ref[r])


_k_in = pl.pallas_call(
    _k_in_body,
    grid=(NB,),
    in_specs=[
        pl.BlockSpec((BLK, D), lambda i: (i, 0)),
        pl.BlockSpec((D, D), lambda i: (0, 0)),
        pl.BlockSpec((1, D), lambda i: (0, 0)),
        pl.BlockSpec((R + 1, D, D), lambda i: (0, 0, 0)),
    ],
    out_specs=pl.BlockSpec((R + 1, BLK, D), lambda i: (0, i, 0)),
    out_shape=jax.ShapeDtypeStruct((R + 1, N, D), jnp.float32),
)


def _k_comb_body(root_ref, a0_ref, a1_ref, b_ref, wcat_ref, out_ref):
    h = jnp.maximum(
        root_ref[...] + a0_ref[...] + a1_ref[...] + b_ref[0][None, :], 0.0)
    for r in range(R + 1):
        out_ref[r] = _mm(h, wcat_ref[r])


_k_comb = pl.pallas_call(
    _k_comb_body,
    grid=(NB,),
    in_specs=[
        pl.BlockSpec((BLK, D), lambda i: (i, 0)),
        pl.BlockSpec((BLK, D), lambda i: (i, 0)),
        pl.BlockSpec((BLK, D), lambda i: (i, 0)),
        pl.BlockSpec((1, D), lambda i: (0, 0)),
        pl.BlockSpec((R + 1, D, D), lambda i: (0, 0, 0)),
    ],
    out_specs=pl.BlockSpec((R + 1, BLK, D), lambda i: (0, i, 0)),
    out_shape=jax.ShapeDtypeStruct((R + 1, N, D), jnp.float32),
)


def _k_mlp_body(root_ref, a0_ref, a1_ref, b_ref,
                wo1_ref, bo1_ref, wo2_ref, bo2_ref, wo3_ref, bo3_ref,
                out_ref):
    h = jnp.maximum(
        root_ref[...] + a0_ref[...] + a1_ref[...] + b_ref[0][None, :], 0.0)
    o = jnp.maximum(_mm(h, wo1_ref[...]) + bo1_ref[0][None, :], 0.0)
    o = jnp.maximum(_mm(o, wo2_ref[...]) + bo2_ref[0][None, :], 0.0)
    out_ref[...] = _mm(o, wo3_ref[...]) + bo3_ref[0][None, :]


_k_mlp = pl.pallas_call(
    _k_mlp_body,
    grid=(NB,),
    in_specs=[
        pl.BlockSpec((BLK, D), lambda i: (i, 0)),
        pl.BlockSpec((BLK, D), lambda i: (i, 0)),
        pl.BlockSpec((BLK, D), lambda i: (i, 0)),
        pl.BlockSpec((1, D), lambda i: (0, 0)),
        pl.BlockSpec((D, 512), lambda i: (0, 0)),
        pl.BlockSpec((1, 512), lambda i: (0, 0)),
        pl.BlockSpec((512, 256), lambda i: (0, 0)),
        pl.BlockSpec((1, 256), lambda i: (0, 0)),
        pl.BlockSpec((256, 128), lambda i: (0, 0)),
        pl.BlockSpec((1, 128), lambda i: (0, 0)),
    ],
    out_specs=pl.BlockSpec((BLK, 128), lambda i: (i, 0)),
    out_shape=jax.ShapeDtypeStruct((N, 128), jnp.float32),
)


def kernel(x, edge_index, edge_type, W_in, b_in, w1_rel, w1_root, b1,
           w2_rel, w2_root, b2, w3_rel, w3_root, b3,
           Wo1, bo1, Wo2, bo2, Wo3, bo3):
    src = edge_index[0]
    dst = edge_index[1]

    cnt2, g, c = _pass_a(src, dst, edge_type)
    w = _pass_b(cnt2, c)

    wcat1 = jnp.concatenate([w1_rel, w1_root[None]], axis=0)
    wcat2 = jnp.concatenate([w2_rel, w2_root[None]], axis=0)
    wcat3 = jnp.concatenate([w3_rel, w3_root[None]], axis=0)

    y = _k_in(x, W_in, b_in.reshape(1, D), wcat1)
    acc = _pass_m(y.reshape((R + 1) * N, D), g, dst, w)
    y = _k_comb(y[R], acc[0], acc[1], b1.reshape(1, D), wcat2)
    acc = _pass_m(y.reshape((R + 1) * N, D), g, dst, w)
    y = _k_comb(y[R], acc[0], acc[1], b2.reshape(1, D), wcat3)
    acc = _pass_m(y.reshape((R + 1) * N, D), g, dst, w)

    wo3p = jnp.pad(Wo3, ((0, 0), (0, 128 - Wo3.shape[1])))
    bo3p = jnp.pad(bo3, (0, 128 - bo3.shape[0]))
    out = _k_mlp(y[R], acc[0], acc[1], b3.reshape(1, D),
                 Wo1, bo1.reshape(1, 512), Wo2, bo2.reshape(1, 256),
                 wo3p, bo3p.reshape(1, 128))
    return out[:, :Wo3.shape[1]]
